# baseline (speedup 1.0000x reference)
import functools

import jax
import jax.numpy as jnp
from jax.experimental import pallas as pl
from jax.experimental.pallas import tpu as pltpu

_KB = 4096
_TOPK = 5
_DEPTH = 4
_NEG = -1e30
_NL = 128
_RG = 512


def _retrieve_body(q_ref, w_ref, b_ref, keys_ref, vals_ref, idx_ref, qn_ref,
                   sv_ref, si_ref, *, n_keys, n_blocks):
    step = pl.program_id(0)
    nq = q_ref.shape[0]

    @pl.when(step == 0)
    def _init():
        q = jnp.dot(q_ref[...], w_ref[...], preferred_element_type=jnp.float32)
        q = q + b_ref[...]
        nrm = jnp.sqrt(jnp.sum(q * q, axis=1, keepdims=True))
        qn_ref[...] = q / jnp.maximum(nrm, 1e-12)
        sv_ref[...] = jnp.full(sv_ref.shape, _NEG, jnp.float32)
        si_ref[...] = jnp.zeros(si_ref.shape, jnp.int32)

    kblk = keys_ref[...]
    ss = jnp.sum(kblk * kblk, axis=1, keepdims=True)
    inv = 1.0 / jnp.maximum(jnp.sqrt(ss), 1e-12)
    riota = jax.lax.broadcasted_iota(jnp.int32, (_KB, 1), 0)
    inv = jnp.where(riota < n_keys - step * _KB, inv, 0.0)
    kn = kblk * inv
    qn = qn_ref[...]

    base = step * _KB
    lane128 = jax.lax.broadcasted_iota(jnp.int32, (_RG, _NL), 1)
    lanes = jax.lax.broadcasted_iota(jnp.int32, (_RG, _DEPTH * _NL), 1)

    s_parts = [
        jax.lax.dot_general(qn[r * _RG:(r + 1) * _RG], kn,
                            (((1,), (1,)), ((), ())),
                            preferred_element_type=jnp.float32)
        for r in range(nq // _RG)
    ]

    for r in range(nq // _RG):
        s = s_parts[r]
        rows = pl.ds(r * _RG, _RG)

        sv = sv_ref[rows, :]
        si = si_ref[rows, :]
        v = [sv[:, d * _NL:(d + 1) * _NL] for d in range(_DEPTH)]
        i = [si[:, d * _NL:(d + 1) * _NL] for d in range(_DEPTH)]
        v1, v2, v3, v4 = v
        i1, i2, i3, i4 = i
        for c in range(_KB // _NL):
            x = s[:, c * _NL:(c + 1) * _NL]
            xi = (base + c * _NL) + lane128
            g1 = x > v1
            g2 = x > v2
            g3 = x > v3
            g4 = x > v4
            v4 = jnp.where(g3, v3, jnp.where(g4, x, v4))
            i4 = jnp.where(g3, i3, jnp.where(g4, xi, i4))
            v3 = jnp.where(g2, v2, jnp.where(g3, x, v3))
            i3 = jnp.where(g2, i2, jnp.where(g3, xi, i3))
            v2 = jnp.where(g1, v1, jnp.where(g2, x, v2))
            i2 = jnp.where(g1, i1, jnp.where(g2, xi, i2))
            v1 = jnp.where(g1, x, v1)
            i1 = jnp.where(g1, xi, i1)
        sv_ref[rows, :] = jnp.concatenate([v1, v2, v3, v4], axis=1)
        si_ref[rows, :] = jnp.concatenate([i1, i2, i3, i4], axis=1)

    @pl.when(step == n_blocks - 1)
    def _final_merge():
        for r in range(nq // _RG):
            rows = pl.ds(r * _RG, _RG)
            cand_i = si_ref[rows, :]
            cv = jnp.where(cand_i < n_keys, sv_ref[rows, :], _NEG)
            nv, ni = [], []
            for _ in range(_TOPK):
                a = jnp.argmax(cv, axis=1).astype(jnp.int32)[:, None]
                oh = lanes == a
                nv.append(jnp.max(cv, axis=1, keepdims=True))
                ni.append(jnp.sum(jnp.where(oh, cand_i, 0), axis=1,
                                  keepdims=True))
                cv = jnp.where(oh, _NEG, cv)
            vals_ref[rows, :] = jnp.concatenate(nv, axis=1)
            idx_ref[rows, :] = jnp.concatenate(ni, axis=1)


def kernel(queries, keys, W, b, k):
    del k
    n_keys, d = keys.shape
    nq, d_in = queries.shape
    n_blocks = pl.cdiv(n_keys, _KB)
    b2 = b.reshape(1, d)
    body = functools.partial(_retrieve_body, n_keys=n_keys, n_blocks=n_blocks)
    vals, idx = pl.pallas_call(
        body,
        grid=(n_blocks,),
        in_specs=[
            pl.BlockSpec((nq, d_in), lambda i: (0, 0)),
            pl.BlockSpec((d_in, d), lambda i: (0, 0)),
            pl.BlockSpec((1, d), lambda i: (0, 0)),
            pl.BlockSpec((_KB, d), lambda i: (i, 0)),
        ],
        out_specs=[
            pl.BlockSpec((nq, _TOPK), lambda i: (0, 0)),
            pl.BlockSpec((nq, _TOPK), lambda i: (0, 0)),
        ],
        out_shape=[
            jax.ShapeDtypeStruct((nq, _TOPK), jnp.float32),
            jax.ShapeDtypeStruct((nq, _TOPK), jnp.int32),
        ],
        scratch_shapes=[
            pltpu.VMEM((nq, d), jnp.float32),
            pltpu.VMEM((nq, _DEPTH * _NL), jnp.float32),
            pltpu.VMEM((nq, _DEPTH * _NL), jnp.int32),
        ],
    )(queries, W, b2, keys)
    return (vals, idx)


# speedup vs baseline: 1.0320x; 1.0320x over previous
import functools

import jax
import jax.numpy as jnp
from jax.experimental import pallas as pl
from jax.experimental.pallas import tpu as pltpu

_KB = 4096
_TOPK = 5
_DEPTH = 4
_NEG = -1e30
_NL = 128
_RG = 256


def _retrieve_body(q_ref, w_ref, b_ref, keys_ref, vals_ref, idx_ref, qn_ref,
                   sv_ref, si_ref, *, n_keys, n_blocks):
    step = pl.program_id(0)
    nq = q_ref.shape[0]

    @pl.when(step == 0)
    def _init():
        q = jnp.dot(q_ref[...], w_ref[...], preferred_element_type=jnp.float32)
        q = q + b_ref[...]
        nrm = jnp.sqrt(jnp.sum(q * q, axis=1, keepdims=True))
        qn_ref[...] = q / jnp.maximum(nrm, 1e-12)
        sv_ref[...] = jnp.full(sv_ref.shape, _NEG, jnp.float32)
        si_ref[...] = jnp.zeros(si_ref.shape, jnp.int32)

    kblk = keys_ref[...]
    ss = jnp.sum(kblk * kblk, axis=1, keepdims=True)
    inv = 1.0 / jnp.maximum(jnp.sqrt(ss), 1e-12)
    riota = jax.lax.broadcasted_iota(jnp.int32, (_KB, 1), 0)
    inv = jnp.where(riota < n_keys - step * _KB, inv, 0.0)
    kn = kblk * inv
    qn = qn_ref[...]

    base = step * _KB
    lane128 = jax.lax.broadcasted_iota(jnp.int32, (_RG, _NL), 1)
    lanes = jax.lax.broadcasted_iota(jnp.int32, (_RG, _DEPTH * _NL), 1)

    s_parts = [
        jax.lax.dot_general(qn[r * _RG:(r + 1) * _RG], kn,
                            (((1,), (1,)), ((), ())),
                            preferred_element_type=jnp.float32)
        for r in range(nq // _RG)
    ]

    for r in range(nq // _RG):
        s = s_parts[r]
        rows = pl.ds(r * _RG, _RG)

        sv = sv_ref[rows, :]
        si = si_ref[rows, :]
        v = [sv[:, d * _NL:(d + 1) * _NL] for d in range(_DEPTH)]
        i = [si[:, d * _NL:(d + 1) * _NL] for d in range(_DEPTH)]
        v1, v2, v3, v4 = v
        i1, i2, i3, i4 = i
        for c in range(_KB // _NL):
            x = s[:, c * _NL:(c + 1) * _NL]
            xi = (base + c * _NL) + lane128
            g1 = x > v1
            g2 = x > v2
            g3 = x > v3
            g4 = x > v4
            v4 = jnp.where(g3, v3, jnp.where(g4, x, v4))
            i4 = jnp.where(g3, i3, jnp.where(g4, xi, i4))
            v3 = jnp.where(g2, v2, jnp.where(g3, x, v3))
            i3 = jnp.where(g2, i2, jnp.where(g3, xi, i3))
            v2 = jnp.where(g1, v1, jnp.where(g2, x, v2))
            i2 = jnp.where(g1, i1, jnp.where(g2, xi, i2))
            v1 = jnp.where(g1, x, v1)
            i1 = jnp.where(g1, xi, i1)
        sv_ref[rows, :] = jnp.concatenate([v1, v2, v3, v4], axis=1)
        si_ref[rows, :] = jnp.concatenate([i1, i2, i3, i4], axis=1)

    @pl.when(step == n_blocks - 1)
    def _final_merge():
        for r in range(nq // _RG):
            rows = pl.ds(r * _RG, _RG)
            cand_i = si_ref[rows, :]
            cv = jnp.where(cand_i < n_keys, sv_ref[rows, :], _NEG)
            nv, ni = [], []
            for _ in range(_TOPK):
                a = jnp.argmax(cv, axis=1).astype(jnp.int32)[:, None]
                oh = lanes == a
                nv.append(jnp.max(cv, axis=1, keepdims=True))
                ni.append(jnp.sum(jnp.where(oh, cand_i, 0), axis=1,
                                  keepdims=True))
                cv = jnp.where(oh, _NEG, cv)
            vals_ref[rows, :] = jnp.concatenate(nv, axis=1)
            idx_ref[rows, :] = jnp.concatenate(ni, axis=1)


def kernel(queries, keys, W, b, k):
    del k
    n_keys, d = keys.shape
    nq, d_in = queries.shape
    n_blocks = pl.cdiv(n_keys, _KB)
    b2 = b.reshape(1, d)
    body = functools.partial(_retrieve_body, n_keys=n_keys, n_blocks=n_blocks)
    vals, idx = pl.pallas_call(
        body,
        grid=(n_blocks,),
        in_specs=[
            pl.BlockSpec((nq, d_in), lambda i: (0, 0)),
            pl.BlockSpec((d_in, d), lambda i: (0, 0)),
            pl.BlockSpec((1, d), lambda i: (0, 0)),
            pl.BlockSpec((_KB, d), lambda i: (i, 0)),
        ],
        out_specs=[
            pl.BlockSpec((nq, _TOPK), lambda i: (0, 0)),
            pl.BlockSpec((nq, _TOPK), lambda i: (0, 0)),
        ],
        out_shape=[
            jax.ShapeDtypeStruct((nq, _TOPK), jnp.float32),
            jax.ShapeDtypeStruct((nq, _TOPK), jnp.int32),
        ],
        scratch_shapes=[
            pltpu.VMEM((nq, d), jnp.float32),
            pltpu.VMEM((nq, _DEPTH * _NL), jnp.float32),
            pltpu.VMEM((nq, _DEPTH * _NL), jnp.int32),
        ],
    )(queries, W, b2, keys)
    return (vals, idx)


# speedup vs baseline: 1.0362x; 1.0040x over previous
import functools

import jax
import jax.numpy as jnp
from jax.experimental import pallas as pl
from jax.experimental.pallas import tpu as pltpu

_KB = 5120
_TOPK = 5
_DEPTH = 4
_NEG = -1e30
_NL = 128
_RG = 256


def _retrieve_body(q_ref, w_ref, b_ref, keys_ref, vals_ref, idx_ref, qn_ref,
                   sv_ref, si_ref, *, n_keys, n_blocks):
    step = pl.program_id(0)
    nq = q_ref.shape[0]

    @pl.when(step == 0)
    def _init():
        q = jnp.dot(q_ref[...], w_ref[...], preferred_element_type=jnp.float32)
        q = q + b_ref[...]
        nrm = jnp.sqrt(jnp.sum(q * q, axis=1, keepdims=True))
        qn_ref[...] = q / jnp.maximum(nrm, 1e-12)
        sv_ref[...] = jnp.full(sv_ref.shape, _NEG, jnp.float32)
        si_ref[...] = jnp.zeros(si_ref.shape, jnp.int32)

    kblk = keys_ref[...]
    ss = jnp.sum(kblk * kblk, axis=1, keepdims=True)
    inv = 1.0 / jnp.maximum(jnp.sqrt(ss), 1e-12)
    riota = jax.lax.broadcasted_iota(jnp.int32, (_KB, 1), 0)
    inv = jnp.where(riota < n_keys - step * _KB, inv, 0.0)
    kn = kblk * inv
    qn = qn_ref[...]

    base = step * _KB
    lane128 = jax.lax.broadcasted_iota(jnp.int32, (_RG, _NL), 1)
    lanes = jax.lax.broadcasted_iota(jnp.int32, (_RG, _DEPTH * _NL), 1)

    s_parts = [
        jax.lax.dot_general(qn[r * _RG:(r + 1) * _RG], kn,
                            (((1,), (1,)), ((), ())),
                            preferred_element_type=jnp.float32)
        for r in range(nq // _RG)
    ]

    for r in range(nq // _RG):
        s = s_parts[r]
        rows = pl.ds(r * _RG, _RG)

        sv = sv_ref[rows, :]
        si = si_ref[rows, :]
        v = [sv[:, d * _NL:(d + 1) * _NL] for d in range(_DEPTH)]
        i = [si[:, d * _NL:(d + 1) * _NL] for d in range(_DEPTH)]
        v1, v2, v3, v4 = v
        i1, i2, i3, i4 = i
        for c in range(_KB // _NL):
            x = s[:, c * _NL:(c + 1) * _NL]
            xi = (base + c * _NL) + lane128
            g1 = x > v1
            g2 = x > v2
            g3 = x > v3
            g4 = x > v4
            v4 = jnp.where(g3, v3, jnp.where(g4, x, v4))
            i4 = jnp.where(g3, i3, jnp.where(g4, xi, i4))
            v3 = jnp.where(g2, v2, jnp.where(g3, x, v3))
            i3 = jnp.where(g2, i2, jnp.where(g3, xi, i3))
            v2 = jnp.where(g1, v1, jnp.where(g2, x, v2))
            i2 = jnp.where(g1, i1, jnp.where(g2, xi, i2))
            v1 = jnp.where(g1, x, v1)
            i1 = jnp.where(g1, xi, i1)
        sv_ref[rows, :] = jnp.concatenate([v1, v2, v3, v4], axis=1)
        si_ref[rows, :] = jnp.concatenate([i1, i2, i3, i4], axis=1)

    @pl.when(step == n_blocks - 1)
    def _final_merge():
        for r in range(nq // _RG):
            rows = pl.ds(r * _RG, _RG)
            cand_i = si_ref[rows, :]
            cv = jnp.where(cand_i < n_keys, sv_ref[rows, :], _NEG)
            nv, ni = [], []
            for _ in range(_TOPK):
                a = jnp.argmax(cv, axis=1).astype(jnp.int32)[:, None]
                oh = lanes == a
                nv.append(jnp.max(cv, axis=1, keepdims=True))
                ni.append(jnp.sum(jnp.where(oh, cand_i, 0), axis=1,
                                  keepdims=True))
                cv = jnp.where(oh, _NEG, cv)
            vals_ref[rows, :] = jnp.concatenate(nv, axis=1)
            idx_ref[rows, :] = jnp.concatenate(ni, axis=1)


def kernel(queries, keys, W, b, k):
    del k
    n_keys, d = keys.shape
    nq, d_in = queries.shape
    n_blocks = pl.cdiv(n_keys, _KB)
    b2 = b.reshape(1, d)
    body = functools.partial(_retrieve_body, n_keys=n_keys, n_blocks=n_blocks)
    vals, idx = pl.pallas_call(
        body,
        grid=(n_blocks,),
        in_specs=[
            pl.BlockSpec((nq, d_in), lambda i: (0, 0)),
            pl.BlockSpec((d_in, d), lambda i: (0, 0)),
            pl.BlockSpec((1, d), lambda i: (0, 0)),
            pl.BlockSpec((_KB, d), lambda i: (i, 0)),
        ],
        out_specs=[
            pl.BlockSpec((nq, _TOPK), lambda i: (0, 0)),
            pl.BlockSpec((nq, _TOPK), lambda i: (0, 0)),
        ],
        out_shape=[
            jax.ShapeDtypeStruct((nq, _TOPK), jnp.float32),
            jax.ShapeDtypeStruct((nq, _TOPK), jnp.int32),
        ],
        scratch_shapes=[
            pltpu.VMEM((nq, d), jnp.float32),
            pltpu.VMEM((nq, _DEPTH * _NL), jnp.float32),
            pltpu.VMEM((nq, _DEPTH * _NL), jnp.int32),
        ],
    )(queries, W, b2, keys)
    return (vals, idx)
